# baseline (speedup 1.0000x reference)
import functools

import jax
import jax.numpy as jnp
from jax import lax
from jax.experimental import pallas as pl
from jax.experimental.pallas import tpu as pltpu
from jax.experimental.pallas import tpu_sc as plsc

NUM_B = 16384
SEQ = 50
DIM = 64
NC, NS = 2, 16
NW = NC * NS
GSZ = 128
N_GROUPS = SEQ * (NUM_B // GSZ)
G_PER_W = N_GROUPS // NW
IDX_PER_W = G_PER_W * GSZ


def _body(idx_hbm, table_hbm, out_hbm, idx_v, rows_v, tiles_v,
          sem_g, sem_w0, sem_w1):
    tbl = table_hbm
    wid = lax.axis_index("s") * NC + lax.axis_index("c")
    g_base = wid * G_PER_W
    sems_w = (sem_w0, sem_w1)

    pltpu.sync_copy(idx_hbm.at[pl.ds(wid * IDX_PER_W, IDX_PER_W)], idx_v)

    def gather(gl, b):
        return pltpu.make_async_copy(
            tbl.at[idx_v.at[pl.ds(gl * GSZ, GSZ)]], rows_v.at[b], sem_g)

    def wb_copy(b, t, s, bh):
        return pltpu.make_async_copy(
            tiles_v.at[b, t], out_hbm.at[s, t, bh], sems_w[b])

    gather(0, 0).start()

    @pl.loop(0, G_PER_W // 2)
    def _pair(p):
        for b in range(2):
            gl = 2 * p + b
            g = g_base + gl
            s = lax.shift_right_logical(g, 7)
            bh = lax.bitwise_and(g, 127)
            gather(gl, b).wait()

            @pl.when(gl + 1 < G_PER_W)
            def _prefetch():
                gather(gl + 1, 1 - b).start()

            @pl.when(gl >= 2)
            def _drain_prev():
                for _ in range(8):
                    wb_copy(b, 0, s, bh).wait()

            rows = rows_v.at[b]

            @pl.loop(0, 8)
            def _tile(t):
                col_base = lax.mul(t, 8)
                for dl in range(8):
                    col = jnp.broadcast_to(col_base + dl, (16,))
                    for j in range(8):
                        rid = lax.iota(jnp.int32, 16) + (16 * j)
                        vec = plsc.load_gather(rows, [rid, col])
                        tiles_v[b, t, dl, pl.ds(16 * j, 16)] = vec
                wb_copy(b, t, s, bh).start()

    for b in range(2):
        for _ in range(8):
            wb_copy(b, 0, 0, 0).wait()


@jax.jit
def _gather(table, idx):
    mesh = plsc.VectorSubcoreMesh(core_axis_name="c", subcore_axis_name="s")
    f = functools.partial(
        pl.kernel,
        out_type=jax.ShapeDtypeStruct((SEQ, 8, NUM_B // GSZ, 8, GSZ),
                                      jnp.float32),
        mesh=mesh,
        scratch_types=[
            pltpu.VMEM((IDX_PER_W,), jnp.int32),
            pltpu.VMEM((2, GSZ, DIM), jnp.float32),
            pltpu.VMEM((2, 8, 8, GSZ), jnp.float32),
            pltpu.SemaphoreType.DMA,
            pltpu.SemaphoreType.DMA,
            pltpu.SemaphoreType.DMA,
        ],
        compiler_params=pltpu.CompilerParams(use_tc_tiling_on_sc=False,
                                             needs_layout_passes=False),
    )(_body)
    return f(idx, table)


def kernel(entity_indices, table):
    idx = entity_indices.T.reshape(-1).astype(jnp.int32)
    out5 = _gather(table, idx)
    return out5.transpose(2, 4, 0, 1, 3).reshape(NUM_B, SEQ, DIM)


# speedup vs baseline: 1.0027x; 1.0027x over previous
import functools

import jax
import jax.numpy as jnp
from jax import lax
from jax.experimental import pallas as pl
from jax.experimental.pallas import tpu as pltpu
from jax.experimental.pallas import tpu_sc as plsc

NUM_B = 16384
SEQ = 50
DIM = 64
NC, NS = 2, 16
NW = NC * NS
GSZ = 128
N_GROUPS = SEQ * (NUM_B // GSZ)
G_PER_W = N_GROUPS // NW
IDX_PER_W = G_PER_W * GSZ
NRB = 4


def _body(idx_hbm, table_hbm, out_hbm, idx_v, rows_v, tiles_v,
          sem_g, sem_w0, sem_w1):
    wid = lax.axis_index("s") * NC + lax.axis_index("c")
    g_base = wid * G_PER_W
    sems_w = (sem_w0, sem_w1)

    pltpu.sync_copy(idx_hbm.at[pl.ds(wid * IDX_PER_W, IDX_PER_W)],
                    idx_v.at[pl.ds(0, IDX_PER_W)])
    pltpu.sync_copy(idx_hbm.at[pl.ds(0, (NRB - 1) * GSZ)],
                    idx_v.at[pl.ds(IDX_PER_W, (NRB - 1) * GSZ)])

    rids = [lax.iota(jnp.int32, 16) + 16 * j for j in range(8)]

    def gather(gl, buf):
        return pltpu.make_async_copy(
            table_hbm.at[idx_v.at[pl.ds(gl * GSZ, GSZ)]],
            rows_v.at[buf], sem_g)

    def wb_copy(tb, t, s, bh):
        return pltpu.make_async_copy(
            tiles_v.at[tb, t], out_hbm.at[s, t, bh], sems_w[tb])

    for k in range(NRB - 1):
        gather(k, k).start()

    @pl.loop(0, G_PER_W // NRB)
    def _quad(p):
        for q in range(NRB):
            gl = NRB * p + q
            g = g_base + gl
            s = lax.shift_right_logical(g, 7)
            bh = lax.bitwise_and(g, 127)
            tb = q % 2
            gather(gl, q).wait()
            gather(gl + NRB - 1, (q + NRB - 1) % NRB).start()

            def drain_prev():
                for _ in range(8):
                    wb_copy(tb, 0, s, bh).wait()

            if q >= 2:
                drain_prev()
            else:
                pl.when(p >= 1)(drain_prev)

            rows = rows_v.at[q]

            @pl.loop(0, 8)
            def _tile(t):
                base_col = lax.mul(t, 8)
                for dl in range(8):
                    col = jnp.broadcast_to(base_col + dl, (16,))
                    for j in range(8):
                        vec = plsc.load_gather(rows, [rids[j], col])
                        tiles_v[tb, t, dl, pl.ds(16 * j, 16)] = vec
                wb_copy(tb, t, s, bh).start()

    for k in range(NRB - 1):
        gather(0, k).wait()
    for tb in range(2):
        for _ in range(8):
            wb_copy(tb, 0, 0, 0).wait()


@jax.jit
def _gather(table, idx):
    mesh = plsc.VectorSubcoreMesh(core_axis_name="c", subcore_axis_name="s")
    f = functools.partial(
        pl.kernel,
        out_type=jax.ShapeDtypeStruct((SEQ, 8, NUM_B // GSZ, 8, GSZ),
                                      jnp.float32),
        mesh=mesh,
        scratch_types=[
            pltpu.VMEM((IDX_PER_W + (NRB - 1) * GSZ,), jnp.int32),
            pltpu.VMEM((NRB, GSZ, DIM), jnp.float32),
            pltpu.VMEM((2, 8, 8, GSZ), jnp.float32),
            pltpu.SemaphoreType.DMA,
            pltpu.SemaphoreType.DMA,
            pltpu.SemaphoreType.DMA,
        ],
        compiler_params=pltpu.CompilerParams(use_tc_tiling_on_sc=False,
                                             needs_layout_passes=False),
    )(_body)
    return f(idx, table)


def kernel(entity_indices, table):
    idx = entity_indices.T.reshape(-1).astype(jnp.int32)
    out5 = _gather(table, idx)
    return out5.transpose(2, 4, 0, 1, 3).reshape(NUM_B, SEQ, DIM)


# speedup vs baseline: 1.1576x; 1.1545x over previous
import functools

import jax
import jax.numpy as jnp
from jax import lax
from jax.experimental import pallas as pl
from jax.experimental.pallas import tpu as pltpu
from jax.experimental.pallas import tpu_sc as plsc

NUM_B = 16384
SEQ = 50
DIM = 64
NC, NS = 2, 16
NW = NC * NS
GSZ = 128
N_GROUPS = SEQ * (NUM_B // GSZ)
G_PER_W = N_GROUPS // NW
IDX_PER_W = G_PER_W * GSZ
NRB = 4


def _body(idx_hbm, table_hbm, out_hbm, idx_v, rows_v, tiles_v,
          sem_g, sem_w0, sem_w1):
    wid = lax.axis_index("s") * NC + lax.axis_index("c")
    g_base = wid * G_PER_W
    sems_w = (sem_w0, sem_w1)

    pltpu.sync_copy(idx_hbm.at[pl.ds(wid * IDX_PER_W, IDX_PER_W)],
                    idx_v.at[pl.ds(0, IDX_PER_W)])
    pltpu.sync_copy(idx_hbm.at[pl.ds(0, (NRB - 1) * GSZ)],
                    idx_v.at[pl.ds(IDX_PER_W, (NRB - 1) * GSZ)])

    rids = [lax.iota(jnp.int32, 16) + 16 * j for j in range(8)]

    def gather(gl, buf):
        return pltpu.make_async_copy(
            table_hbm.at[idx_v.at[pl.ds(gl * GSZ, GSZ)]],
            rows_v.at[buf], sem_g)

    def wb_copy(tb, t, s, bh):
        return pltpu.make_async_copy(
            tiles_v.at[tb, t], out_hbm.at[s, t, bh], sems_w[tb])

    for k in range(NRB - 1):
        gather(k, k).start()

    @pl.loop(0, G_PER_W // NRB)
    def _quad(p):
        for q in range(NRB):
            gl = NRB * p + q
            g = g_base + gl
            s = lax.shift_right_logical(g, 7)
            bh = lax.bitwise_and(g, 127)
            tb = q % 2
            gather(gl, q).wait()
            gather(gl + NRB - 1, (q + NRB - 1) % NRB).start()

            def drain_prev():
                for _ in range(8):
                    wb_copy(tb, 0, s, bh).wait()

            if q >= 2:
                drain_prev()
            else:
                pl.when(p >= 1)(drain_prev)

            rows = rows_v.at[q]

            @pl.loop(0, 8)
            def _tile(t):
                base_col = lax.mul(t, 8)
                for dl in range(8):
                    col = jnp.broadcast_to(base_col + dl, (16,))
                    vecs = [plsc.load_gather(rows, [rids[j], col])
                            for j in range(8)]
                    for j in range(8):
                        tiles_v[tb, t, dl, pl.ds(16 * j, 16)] = vecs[j]
                wb_copy(tb, t, s, bh).start()

    for k in range(NRB - 1):
        gather(0, k).wait()
    for tb in range(2):
        for _ in range(8):
            wb_copy(tb, 0, 0, 0).wait()


@jax.jit
def _gather(table, idx):
    mesh = plsc.VectorSubcoreMesh(core_axis_name="c", subcore_axis_name="s")
    f = functools.partial(
        pl.kernel,
        out_type=jax.ShapeDtypeStruct((SEQ, 8, NUM_B // GSZ, 8, GSZ),
                                      jnp.float32),
        mesh=mesh,
        scratch_types=[
            pltpu.VMEM((IDX_PER_W + (NRB - 1) * GSZ,), jnp.int32),
            pltpu.VMEM((NRB, GSZ, DIM), jnp.float32),
            pltpu.VMEM((2, 8, 8, GSZ), jnp.float32),
            pltpu.SemaphoreType.DMA,
            pltpu.SemaphoreType.DMA,
            pltpu.SemaphoreType.DMA,
        ],
        compiler_params=pltpu.CompilerParams(use_tc_tiling_on_sc=False,
                                             needs_layout_passes=False),
    )(_body)
    return f(idx, table)


def kernel(entity_indices, table):
    idx = entity_indices.T.reshape(-1).astype(jnp.int32)
    out5 = _gather(table, idx)
    return out5.transpose(2, 4, 0, 1, 3).reshape(NUM_B, SEQ, DIM)
